# baseline (speedup 1.0000x reference)
import functools

import jax
import jax.numpy as jnp
from jax.experimental import pallas as pl
from jax.experimental.pallas import tpu as pltpu

EMB = 100
N_NODE = 50000
LAYERS = 3
B = 512
L = 50

VPAD = 51200
VCHUNK = 3200


def _scores_body(select_ref, table_ref, scores_ref, m_ref, s_ref):
    j = pl.program_id(0)
    sel = select_ref[...]
    blk = jnp.dot(sel, table_ref[...].T, preferred_element_type=jnp.float32)
    scores_ref[...] = blk
    blk = blk + 1e-8
    col = j * VCHUNK + jax.lax.broadcasted_iota(jnp.int32, blk.shape, 1)
    blk = jnp.where(col < N_NODE, blk, -1e30)
    bmax = jnp.max(blk, axis=1)

    @pl.when(j == 0)
    def _init():
        m_ref[...] = jnp.full_like(m_ref, -jnp.inf)
        s_ref[...] = jnp.zeros_like(s_ref)

    m_old = m_ref[...]
    m_new = jnp.maximum(m_old, bmax)
    scale = jnp.exp(m_old - m_new)
    bsum = jnp.sum(jnp.exp(blk - m_new[:, None]), axis=1)
    s_ref[...] = s_ref[...] * scale + bsum
    m_ref[...] = m_new


def _scores_pallas(select, table1):
    table1 = jnp.pad(table1, ((0, VPAD - N_NODE), (0, 0)))
    nblocks = VPAD // VCHUNK
    scores, m, s = pl.pallas_call(
        _scores_body,
        grid=(nblocks,),
        in_specs=[
            pl.BlockSpec((B, EMB), lambda j: (0, 0)),
            pl.BlockSpec((VCHUNK, EMB), lambda j: (j, 0)),
        ],
        out_specs=[
            pl.BlockSpec((B, VCHUNK), lambda j: (0, j)),
            pl.BlockSpec((B,), lambda j: (0,)),
            pl.BlockSpec((B,), lambda j: (0,)),
        ],
        out_shape=[
            jax.ShapeDtypeStruct((B, VPAD), jnp.float32),
            jax.ShapeDtypeStruct((B,), jnp.float32),
            jax.ShapeDtypeStruct((B,), jnp.float32),
        ],
    )(select, table1)
    logZ = m + jnp.log(s)
    return scores[:, :N_NODE], logZ


def _spmm(edge_index, edge_weight, x, n):
    src = edge_index[0]
    dst = edge_index[1]
    msgs = x[src] * edge_weight[:, None]
    return jax.ops.segment_sum(msgs, dst, num_segments=n)


def _soft_attention(seq_h, mask, session_len, pos_emb, w1_W, w1_b, glu1_W,
                    glu1_b, glu2_W, w_2):
    hs = jnp.sum(seq_h, axis=1) / session_len
    maskf = mask[..., None]
    ln = seq_h.shape[1]
    pe = jnp.broadcast_to(pos_emb[:ln][None, :, :], (seq_h.shape[0], ln, EMB))
    hs2 = jnp.broadcast_to(hs[:, None, :], (seq_h.shape[0], ln, EMB))
    nh = jnp.tanh(jnp.concatenate([pe, seq_h], axis=-1) @ w1_W + w1_b)
    nh = jax.nn.sigmoid(nh @ glu1_W + glu1_b + hs2 @ glu2_W)
    beta = nh @ w_2
    beta = beta * maskf
    return jnp.sum(beta * seq_h, axis=1)


def kernel(tar, session_item, reversed_sess_item, mask, session_len, is_train,
           hg_edge_index, hg_edge_weight, gnn_edge_index, gnn_edge_weight,
           nodes_emb, W_hg, b_hg, W_gnn, b_gnn, pos_emb,
           w_W, w_b, w1_W, w1_b, w_2, glu1_W, glu1_b, glu2_W):
    n = N_NODE + 1
    item_embeddings = jnp.concatenate(
        [jnp.zeros((1, EMB), jnp.float32), nodes_emb], axis=0)

    x = item_embeddings
    acc = x
    for i in range(LAYERS):
        x = _spmm(hg_edge_index, hg_edge_weight, x @ W_hg[i] + b_hg[i], n)
        acc = acc + x
    item_embeddings_hg = acc / (LAYERS + 1)

    x = item_embeddings
    acc = x
    for i in range(LAYERS - 1):
        x = _spmm(gnn_edge_index, gnn_edge_weight, x @ W_gnn[i] + b_gnn[i], n)
        acc = acc + x
    item_embeddings_gnn = acc / LAYERS

    seq_h_hg = item_embeddings_hg[reversed_sess_item]
    seq_h_gnn = item_embeddings_gnn[reversed_sess_item]
    select_hg = _soft_attention(seq_h_hg, mask, session_len, pos_emb, w1_W,
                                w1_b, glu1_W, glu1_b, glu2_W, w_2)
    select_gnn = _soft_attention(seq_h_gnn, mask, session_len, pos_emb, w1_W,
                                 w1_b, glu1_W, glu1_b, glu2_W, w_2)
    select = jnp.concatenate([select_hg, select_gnn], axis=-1) @ w_W + w_b

    scores, logZ = _scores_pallas(select, item_embeddings[1:])
    logits_tar = jnp.take_along_axis(scores + 1e-8, tar[:, None], axis=1)[:, 0]
    nll = logZ - logits_tar
    loss = jnp.mean(nll)
    con_loss = jnp.float32(0.0)
    return (con_loss, loss, scores)


# speedup vs baseline: 1.0641x; 1.0641x over previous
import functools

import jax
import jax.numpy as jnp
from jax import lax
from jax.experimental import pallas as pl
from jax.experimental.pallas import tpu as pltpu
from jax.experimental.pallas import tpu_sc as plsc

EMB = 100
N_NODE = 50000
LAYERS = 3
B = 512
L = 50
E = 800000

D = 128
NW = 32
NR = 64
RPR = 784
NPAD = NR * RPR
FCH = 4096
NCHF = (E + FCH - 1) // FCH
EPADF = NCHF * FCH
FL = 2048
LCAP = EPADF + FL
CH = 128
BL = B * L
PW = BL // NW
VPAD = 51200
VCHUNK = 3200

_SC_MESH = dict(core_axis_name="c", subcore_axis_name="s")
_SC_PARAMS = pltpu.CompilerParams(needs_layout_passes=False)


def _filter_body(srcH, dstH, wH, lsrcH, lwH, ldlH, cntH,
                 sv, dv, wv, s0, d0, w0, s1, d1, w1, cs, sem):
    wid = lax.axis_index("s") * 2 + lax.axis_index("c")
    r0 = wid * 2
    lo = wid * (2 * RPR)

    def flush(bufs, rr):
        def do(carry):
            bb, hh = carry
            off = pl.multiple_of(rr * LCAP + hh, 8)
            pltpu.sync_copy(bufs[0].at[pl.ds(0, FL)], lsrcH.at[pl.ds(off, FL)])
            pltpu.sync_copy(bufs[1].at[pl.ds(0, FL)], ldlH.at[pl.ds(off, FL)])
            pltpu.sync_copy(bufs[2].at[pl.ds(0, FL)], lwH.at[pl.ds(off, FL)])
            for bf in bufs:
                t = bf[pl.ds(FL, 16)]
                bf[pl.ds(0, 16)] = t
            return bb - FL, hh + FL
        return do

    def cbody(c, carry):
        base = c * FCH
        pltpu.sync_copy(srcH.at[pl.ds(base, FCH)], sv)
        pltpu.sync_copy(dstH.at[pl.ds(base, FCH)], dv)
        pltpu.sync_copy(wH.at[pl.ds(base, FCH)], wv)

        def gbody(g, carry):
            b0, h0, b1, h1 = carry
            s = sv[pl.ds(g * 16, 16)]
            d = dv[pl.ds(g * 16, 16)]
            w = wv[pl.ds(g * 16, 16)]
            dl = d - lo
            m0 = (dl >= 0) & (dl < RPR)
            dl1 = dl - RPR
            m1 = (dl1 >= 0) & (dl1 < RPR)
            plsc.store_compressed(s0.at[pl.ds(b0, 16)], s, mask=m0)
            plsc.store_compressed(d0.at[pl.ds(b0, 16)], dl, mask=m0)
            plsc.store_compressed(w0.at[pl.ds(b0, 16)], w, mask=m0)
            plsc.store_compressed(s1.at[pl.ds(b1, 16)], s, mask=m1)
            plsc.store_compressed(d1.at[pl.ds(b1, 16)], dl1, mask=m1)
            plsc.store_compressed(w1.at[pl.ds(b1, 16)], w, mask=m1)
            b0 = b0 + plsc.all_reduce_population_count(m0)[0]
            b1 = b1 + plsc.all_reduce_population_count(m1)[0]
            b0, h0 = lax.cond(b0 >= FL, flush((s0, d0, w0), r0),
                              lambda cr: cr, (b0, h0))
            b1, h1 = lax.cond(b1 >= FL, flush((s1, d1, w1), r0 + 1),
                              lambda cr: cr, (b1, h1))
            return b0, h0, b1, h1

        return lax.fori_loop(0, FCH // 16, gbody, carry)

    z = jnp.int32(0)
    b0, h0, b1, h1 = lax.fori_loop(0, NCHF, cbody, (z, z, z, z))
    _, _ = flush((s0, d0, w0), r0)((b0, h0))
    _, _ = flush((s1, d1, w1), r0 + 1)((b1, h1))
    cs[pl.ds(0, 16)] = jnp.full((16,), h0 + b0, jnp.int32)
    pltpu.sync_copy(cs.at[pl.ds(0, 8)], cntH.at[pl.ds(r0 * 8, 8)])
    cs[pl.ds(0, 16)] = jnp.full((16,), h1 + b1, jnp.int32)
    pltpu.sync_copy(cs.at[pl.ds(0, 8)], cntH.at[pl.ds((r0 + 1) * 8, 8)])


def _make_filter():
    return functools.partial(
        pl.kernel,
        _filter_body,
        out_type=[
            jax.ShapeDtypeStruct((NR * LCAP,), jnp.int32),
            jax.ShapeDtypeStruct((NR * LCAP,), jnp.float32),
            jax.ShapeDtypeStruct((NR * LCAP,), jnp.int32),
            jax.ShapeDtypeStruct((NR * 8,), jnp.int32),
        ],
        mesh=plsc.VectorSubcoreMesh(**_SC_MESH),
        compiler_params=_SC_PARAMS,
        scratch_types=[
            pltpu.VMEM((FCH,), jnp.int32),
            pltpu.VMEM((FCH,), jnp.int32),
            pltpu.VMEM((FCH,), jnp.float32),
            pltpu.VMEM((FL + 16,), jnp.int32),
            pltpu.VMEM((FL + 16,), jnp.int32),
            pltpu.VMEM((FL + 16,), jnp.float32),
            pltpu.VMEM((FL + 16,), jnp.int32),
            pltpu.VMEM((FL + 16,), jnp.int32),
            pltpu.VMEM((FL + 16,), jnp.float32),
            pltpu.VMEM((16,), jnp.int32),
            pltpu.SemaphoreType.DMA,
        ],
    )()


def _spmm_body(lsrcH, lwH, ldlH, cntH, xH, outH,
               idxv, wv, dlv, rows, acc, cs, sem):
    wid = lax.axis_index("s") * 2 + lax.axis_index("c")
    iota = lax.iota(jnp.int32, 16)
    zero = jnp.zeros((16,), jnp.float32)

    def rbody(rr, _):
        r = wid * 2 + rr

        def zb(i, _):
            for kk in range(D // 16):
                acc[i, pl.ds(kk * 16, 16)] = zero
            return 0
        lax.fori_loop(0, RPR, zb, 0)

        pltpu.sync_copy(cntH.at[pl.ds(r * 8, 8)], cs.at[pl.ds(0, 8)])
        cnt = cs[pl.ds(0, 16)][0]
        nsub = lax.div(cnt + (CH - 1), CH)

        def sbody(s, _):
            base = pl.multiple_of(r * LCAP + s * CH, 8)
            pltpu.sync_copy(lsrcH.at[pl.ds(base, CH)], idxv)
            pltpu.sync_copy(lwH.at[pl.ds(base, CH)], wv)
            pltpu.sync_copy(ldlH.at[pl.ds(base, CH)], dlv)
            for g in range(CH // 16):
                v = idxv[pl.ds(g * 16, 16)]
                idxv[pl.ds(g * 16, 16)] = jnp.clip(v, 0, NPAD - 1)
            pltpu.async_copy(xH.at[idxv], rows, sem).wait()

            def gb(g, _):
                dl = dlv[pl.ds(g * 16, 16)]
                w = wv[pl.ds(g * 16, 16)]
                mrem = (s * CH + g * 16 + iota) < cnt
                w = jnp.where(mrem, w, 0.0)
                dl = jnp.clip(dl, 0, RPR - 1)
                for j in range(16):
                    wj = w[j]
                    ridx = jnp.full((16,), dl[j], jnp.int32)
                    e = g * 16 + j
                    for kk in range(D // 16):
                        seg = rows[e, pl.ds(kk * 16, 16)]
                        plsc.addupdate_scatter(
                            acc, [ridx, kk * 16 + iota], seg * wj)
                return 0
            lax.fori_loop(0, CH // 16, gb, 0)
            return 0
        lax.fori_loop(0, nsub, sbody, 0)
        pltpu.sync_copy(acc, outH.at[pl.ds(r * RPR, RPR)])
        return 0
    lax.fori_loop(0, 2, rbody, 0)


def _make_spmm():
    return functools.partial(
        pl.kernel,
        _spmm_body,
        out_type=jax.ShapeDtypeStruct((NPAD, D), jnp.float32),
        mesh=plsc.VectorSubcoreMesh(**_SC_MESH),
        compiler_params=_SC_PARAMS,
        scratch_types=[
            pltpu.VMEM((CH,), jnp.int32),
            pltpu.VMEM((CH,), jnp.float32),
            pltpu.VMEM((CH,), jnp.int32),
            pltpu.VMEM((CH, D), jnp.float32),
            pltpu.VMEM((RPR, D), jnp.float32),
            pltpu.VMEM((16,), jnp.int32),
            pltpu.SemaphoreType.DMA,
        ],
    )()


def _sgather_body(idxH, t0H, t1H, t2H, t3H, o0H, o1H, o2H, o3H,
                  idxv, rows, sem):
    wid = lax.axis_index("s") * 2 + lax.axis_index("c")
    base = wid * PW
    zero = jnp.zeros((16,), jnp.int32)
    tabs = ((t0H, o0H), (t1H, o1H), (t2H, o2H), (t3H, o3H))

    def sbody(s, _):
        off = base + s * CH
        pltpu.sync_copy(idxH.at[pl.ds(off, CH)], idxv)
        for tH, oH in tabs:
            pltpu.async_copy(tH.at[idxv], rows, sem).wait()
            pltpu.sync_copy(rows, oH.at[pl.ds(off, CH)])
        return 0
    lax.fori_loop(0, PW // CH, sbody, 0)
    toff = base + (PW // CH) * CH
    pltpu.sync_copy(idxH.at[pl.ds(toff, 32)], idxv.at[pl.ds(0, 32)])
    for g in range(2, CH // 16):
        idxv[pl.ds(g * 16, 16)] = zero
    for tH, oH in tabs:
        pltpu.async_copy(tH.at[idxv], rows, sem).wait()
        pltpu.sync_copy(rows.at[pl.ds(0, 32)], oH.at[pl.ds(toff, 32)])


def _make_sgather():
    return functools.partial(
        pl.kernel,
        _sgather_body,
        out_type=[jax.ShapeDtypeStruct((BL, D), jnp.float32)
                  for _ in range(4)],
        mesh=plsc.VectorSubcoreMesh(**_SC_MESH),
        compiler_params=_SC_PARAMS,
        scratch_types=[
            pltpu.VMEM((CH,), jnp.int32),
            pltpu.VMEM((CH, D), jnp.float32),
            pltpu.SemaphoreType.DMA,
        ],
    )()


def _dense_body(x_ref, w_ref, b_ref, acc_ref, y_ref, accout_ref):
    x = x_ref[...]
    y_ref[...] = jnp.dot(x, w_ref[...],
                         preferred_element_type=jnp.float32) + b_ref[...]
    accout_ref[...] = acc_ref[...] + x


def _dense(x, w, b, acc):
    return pl.pallas_call(
        _dense_body,
        grid=(NPAD // 512,),
        in_specs=[
            pl.BlockSpec((512, D), lambda i: (i, 0)),
            pl.BlockSpec((D, D), lambda i: (0, 0)),
            pl.BlockSpec((1, D), lambda i: (0, 0)),
            pl.BlockSpec((512, D), lambda i: (i, 0)),
        ],
        out_specs=[
            pl.BlockSpec((512, D), lambda i: (i, 0)),
            pl.BlockSpec((512, D), lambda i: (i, 0)),
        ],
        out_shape=[
            jax.ShapeDtypeStruct((NPAD, D), jnp.float32),
            jax.ShapeDtypeStruct((NPAD, D), jnp.float32),
        ],
    )(x, w, b, acc)


def _attn_body(gAh_ref, gBh_ref, gAg_ref, gBg_ref, mask_ref, slen_ref,
               pos_ref, W1a_ref, W1b_ref, b1_ref, G1_ref, g1b_ref, G2_ref,
               w2_ref, wWa_ref, wWb_ref, wb_ref, sel_ref):
    mask = mask_ref[...]
    slen = slen_ref[...]
    pe = jnp.dot(pos_ref[...], W1a_ref[...],
                 preferred_element_type=jnp.float32) + b1_ref[...]
    peb = jnp.broadcast_to(pe[:L][None], (64, L, D)).reshape(64 * L, D)

    def view(a_ref, b_ref, scale):
        seqf = (a_ref[...] + b_ref[...]) * scale
        seq3 = seqf.reshape(64, L, D)
        hs = jnp.sum(seq3, axis=1) / slen
        nh = jnp.tanh(jnp.dot(seqf, W1b_ref[...],
                              preferred_element_type=jnp.float32) + peb)
        hsg = jnp.dot(hs, G2_ref[...], preferred_element_type=jnp.float32)
        hsb = jnp.broadcast_to(hsg[:, None, :], (64, L, D)).reshape(64 * L, D)
        nh2 = jax.nn.sigmoid(
            jnp.dot(nh, G1_ref[...], preferred_element_type=jnp.float32)
            + g1b_ref[...] + hsb)
        beta = jnp.sum(nh2 * w2_ref[...], axis=1).reshape(64, L)
        beta = beta * mask
        return jnp.sum(beta[:, :, None] * seq3, axis=1)

    sel_hg = view(gAh_ref, gBh_ref, 1.0 / (LAYERS + 1))
    sel_gnn = view(gAg_ref, gBg_ref, 1.0 / LAYERS)
    sel_ref[...] = (
        jnp.dot(sel_hg, wWa_ref[...], preferred_element_type=jnp.float32)
        + jnp.dot(sel_gnn, wWb_ref[...], preferred_element_type=jnp.float32)
        + wb_ref[...])


def _attention(gAh, gBh, gAg, gBg, mask, slen, pos_pad, W1a, W1b, b1,
               G1, g1b, G2, w2, wWa, wWb, wb):
    blk = lambda i: (i, 0)
    inv = lambda i: (0, 0)
    return pl.pallas_call(
        _attn_body,
        grid=(B // 64,),
        in_specs=[
            pl.BlockSpec((64 * L, D), blk),
            pl.BlockSpec((64 * L, D), blk),
            pl.BlockSpec((64 * L, D), blk),
            pl.BlockSpec((64 * L, D), blk),
            pl.BlockSpec((64, L), blk),
            pl.BlockSpec((64, 1), blk),
            pl.BlockSpec((64, D), inv),
            pl.BlockSpec((D, D), inv),
            pl.BlockSpec((D, D), inv),
            pl.BlockSpec((1, D), inv),
            pl.BlockSpec((D, D), inv),
            pl.BlockSpec((1, D), inv),
            pl.BlockSpec((D, D), inv),
            pl.BlockSpec((1, D), inv),
            pl.BlockSpec((D, D), inv),
            pl.BlockSpec((D, D), inv),
            pl.BlockSpec((1, D), inv),
        ],
        out_specs=pl.BlockSpec((64, D), blk),
        out_shape=jax.ShapeDtypeStruct((B, D), jnp.float32),
    )(gAh, gBh, gAg, gBg, mask, slen, pos_pad, W1a, W1b, b1, G1, g1b, G2,
      w2, wWa, wWb, wb)


def _scores_body(select_ref, table_ref, tar_ref, scores_ref, m_ref, s_ref,
                 t_ref):
    j = pl.program_id(0)
    blk = jnp.dot(select_ref[...], table_ref[...].T,
                  preferred_element_type=jnp.float32)
    scores_ref[...] = blk
    logits = blk + 1e-8
    col = j * VCHUNK + lax.broadcasted_iota(jnp.int32, blk.shape, 1)
    masked = jnp.where(col < N_NODE, logits, -1e30)
    bmax = jnp.max(masked, axis=1)

    @pl.when(j == 0)
    def _init():
        m_ref[...] = jnp.full_like(m_ref, -jnp.inf)
        s_ref[...] = jnp.zeros_like(s_ref)
        t_ref[...] = jnp.zeros_like(t_ref)

    m_old = m_ref[...]
    m_new = jnp.maximum(m_old, bmax)
    scale = jnp.exp(m_old - m_new)
    bsum = jnp.sum(jnp.exp(masked - m_new[:, None]), axis=1)
    s_ref[...] = s_ref[...] * scale + bsum
    m_ref[...] = m_new
    pick = col == tar_ref[...][:, None]
    t_ref[...] = t_ref[...] + jnp.sum(jnp.where(pick, logits, 0.0), axis=1)


def _scores(select, table_pad, tar):
    nblocks = VPAD // VCHUNK
    scores, m, s, t = pl.pallas_call(
        _scores_body,
        grid=(nblocks,),
        in_specs=[
            pl.BlockSpec((B, D), lambda j: (0, 0)),
            pl.BlockSpec((VCHUNK, D), lambda j: (j, 0)),
            pl.BlockSpec((B,), lambda j: (0,)),
        ],
        out_specs=[
            pl.BlockSpec((B, VCHUNK), lambda j: (0, j)),
            pl.BlockSpec((B,), lambda j: (0,)),
            pl.BlockSpec((B,), lambda j: (0,)),
            pl.BlockSpec((B,), lambda j: (0,)),
        ],
        out_shape=[
            jax.ShapeDtypeStruct((B, VPAD), jnp.float32),
            jax.ShapeDtypeStruct((B,), jnp.float32),
            jax.ShapeDtypeStruct((B,), jnp.float32),
            jax.ShapeDtypeStruct((B,), jnp.float32),
        ],
    )(select, table_pad, tar)
    logZ = m + jnp.log(s)
    return scores[:, :N_NODE], logZ, t


def _pad_w(w):
    return jnp.pad(w, ((0, D - w.shape[0]), (0, D - w.shape[1])))


def _pad_b(b):
    return jnp.pad(b, (0, D - b.shape[0]))[None]


def _edges(edge_index, edge_weight):
    src = jnp.pad(edge_index[0].astype(jnp.int32), (0, EPADF - E))
    dst = jnp.pad(edge_index[1].astype(jnp.int32), (0, EPADF - E))
    w = jnp.pad(edge_weight, (0, EPADF - E))
    return src, dst, w


def kernel(tar, session_item, reversed_sess_item, mask, session_len, is_train,
           hg_edge_index, hg_edge_weight, gnn_edge_index, gnn_edge_weight,
           nodes_emb, W_hg, b_hg, W_gnn, b_gnn, pos_emb,
           w_W, w_b, w1_W, w1_b, w_2, glu1_W, glu1_b, glu2_W):
    x0 = jnp.pad(nodes_emb, ((1, NPAD - N_NODE - 1), (0, D - EMB)))

    filt = _make_filter()
    spmm = _make_spmm()
    hsrc, hdst, hw = _edges(hg_edge_index, hg_edge_weight)
    gsrc, gdst, gw = _edges(gnn_edge_index, gnn_edge_weight)
    h_ls, h_lw, h_ld, h_cnt = filt(hsrc, hdst, hw)
    g_ls, g_lw, g_ld, g_cnt = filt(gsrc, gdst, gw)

    zeros = jnp.zeros((NPAD, D), jnp.float32)
    y0, acc0 = _dense(x0, _pad_w(W_hg[0]), _pad_b(b_hg[0]), zeros)
    x1 = spmm(h_ls, h_lw, h_ld, h_cnt, y0)
    y1, acc1 = _dense(x1, _pad_w(W_hg[1]), _pad_b(b_hg[1]), acc0)
    x2 = spmm(h_ls, h_lw, h_ld, h_cnt, y1)
    y2, acc2 = _dense(x2, _pad_w(W_hg[2]), _pad_b(b_hg[2]), acc1)
    x3 = spmm(h_ls, h_lw, h_ld, h_cnt, y2)
    yg0, _ = _dense(x0, _pad_w(W_gnn[0]), _pad_b(b_gnn[0]), zeros)
    xg1 = spmm(g_ls, g_lw, g_ld, g_cnt, yg0)
    yg1, accg1 = _dense(xg1, _pad_w(W_gnn[1]), _pad_b(b_gnn[1]), acc0)
    xg2 = spmm(g_ls, g_lw, g_ld, g_cnt, yg1)

    rsi = reversed_sess_item.astype(jnp.int32).reshape(BL)
    sg = _make_sgather()
    gAh, gBh, gAg, gBg = sg(rsi, acc2, x3, accg1, xg2)

    pos_pad = jnp.pad(pos_emb[:64], ((0, 0), (0, D - EMB)))
    select = _attention(
        gAh, gBh, gAg, gBg, mask, session_len, pos_pad,
        _pad_w(w1_W[:EMB]), _pad_w(w1_W[EMB:]), _pad_b(w1_b),
        _pad_w(glu1_W), _pad_b(glu1_b), _pad_w(glu2_W),
        _pad_b(w_2[:, 0]),
        _pad_w(w_W[:EMB]), _pad_w(w_W[EMB:]), _pad_b(w_b))

    table_pad = jnp.pad(nodes_emb, ((0, VPAD - N_NODE), (0, D - EMB)))
    scores, logZ, tlogit = _scores(select, table_pad,
                                   tar.astype(jnp.int32))
    loss = jnp.mean(logZ - tlogit)
    con_loss = jnp.float32(0.0)
    return (con_loss, loss, scores)
